# Initial kernel scaffold; baseline (speedup 1.0000x reference)
import jax
import jax.numpy as jnp
from jax.experimental import pallas as pl


def kernel(cs, cs_p, overpass_mask):
    raise NotImplementedError("write your pallas kernel here")


# speedup vs baseline: 2.7770x; 2.7770x over previous
import jax
import jax.numpy as jnp
from jax.experimental import pallas as pl
from jax.experimental.pallas import tpu as pltpu

B, H, W, L = 8, 90, 256, 256
HB = 10
NH = H // HB


def _mse_body(cs_ref, csp_ref, m_ref, out_ref, acc_ref):
    b = pl.program_id(0)
    h = pl.program_id(1)

    @pl.when((b == 0) & (h == 0))
    def _init():
        acc_ref[...] = jnp.zeros_like(acc_ref)

    a = cs_ref[0]
    p = csp_ref[0]
    m = m_ref[0, 0]
    a = jnp.where(jnp.isnan(a), 0.0, a)
    p = jnp.where(m > 0.0, p, 0.0)
    d = a - p
    acc_ref[...] += jnp.sum(d * d, axis=0)

    @pl.when((b == B - 1) & (h == NH - 1))
    def _fin():
        total = jnp.float32(B * H * W * L)
        out_ref[0, 0] = jnp.sum(acc_ref[...]) / total


def kernel(cs, cs_p, overpass_mask):
    cs4 = cs.reshape(B, H, W, L)
    m4 = overpass_mask.reshape(B, 1, W, L)
    out = pl.pallas_call(
        _mse_body,
        grid=(B, NH),
        in_specs=[
            pl.BlockSpec((1, HB, W, L), lambda b, h: (b, h, 0, 0)),
            pl.BlockSpec((1, HB, W, L), lambda b, h: (b, h, 0, 0)),
            pl.BlockSpec((1, 1, W, L), lambda b, h: (b, 0, 0, 0)),
        ],
        out_specs=pl.BlockSpec(memory_space=pltpu.SMEM),
        out_shape=jax.ShapeDtypeStruct((1, 1), jnp.float32),
        scratch_shapes=[pltpu.VMEM((W, L), jnp.float32)],
    )(cs4, cs_p, m4)
    return out[0, 0]
